# Initial kernel scaffold; baseline (speedup 1.0000x reference)
import jax
import jax.numpy as jnp
from jax.experimental import pallas as pl


def kernel(hidden_states, classifier_weight, e_score_correction_bias):
    raise NotImplementedError("write your pallas kernel here")


# speedup vs baseline: 1.5034x; 1.5034x over previous
import functools

import jax
import jax.numpy as jnp
from jax.experimental import pallas as pl
from jax.experimental.pallas import tpu as pltpu

HIDDEN = 2048
NUM_EXPERTS = 64
TOP_K = 8
ROUTED_SCALING_FACTOR = 1.5

TOKEN_BLOCK = 512


def _router_body(hs_ref, w_ref, bias_ref, idx_ref, wgt_ref):
    hs = hs_ref[...]
    w = w_ref[...]
    bias = bias_ref[...]

    logits = jax.lax.dot_general(
        hs, w, (((1,), (1,)), ((), ())),
        preferred_element_type=jnp.float32)

    m = jnp.max(logits, axis=-1, keepdims=True)
    e = jnp.exp(logits - m)
    probs = e / jnp.sum(e, axis=-1, keepdims=True)

    work = probs + bias
    lane = jax.lax.broadcasted_iota(jnp.int32, work.shape, 1)

    idx_cols = []
    wgt_cols = []
    for _ in range(TOP_K):
        mx = jnp.max(work, axis=-1, keepdims=True)
        hit = work == mx
        amx = jnp.min(jnp.where(hit, lane, NUM_EXPERTS), axis=-1,
                      keepdims=True)
        sel = lane == amx
        wsel = jnp.sum(jnp.where(sel, probs, 0.0), axis=-1, keepdims=True)
        idx_cols.append(amx)
        wgt_cols.append(wsel)
        work = jnp.where(sel, -jnp.inf, work)

    idx_ref[...] = jnp.concatenate(idx_cols, axis=-1)
    wgt_ref[...] = jnp.concatenate(wgt_cols, axis=-1) * ROUTED_SCALING_FACTOR


@jax.jit
def _router(hidden_states, classifier_weight, bias2d):
    n_tokens = hidden_states.shape[0]
    grid = (n_tokens // TOKEN_BLOCK,)
    return pl.pallas_call(
        _router_body,
        grid=grid,
        in_specs=[
            pl.BlockSpec((TOKEN_BLOCK, HIDDEN), lambda i: (i, 0)),
            pl.BlockSpec((NUM_EXPERTS, HIDDEN), lambda i: (0, 0)),
            pl.BlockSpec((1, NUM_EXPERTS), lambda i: (0, 0)),
        ],
        out_specs=[
            pl.BlockSpec((TOKEN_BLOCK, TOP_K), lambda i: (i, 0)),
            pl.BlockSpec((TOKEN_BLOCK, TOP_K), lambda i: (i, 0)),
        ],
        out_shape=[
            jax.ShapeDtypeStruct((n_tokens, TOP_K), jnp.int32),
            jax.ShapeDtypeStruct((n_tokens, TOP_K), jnp.float32),
        ],
    )(hidden_states, classifier_weight, bias2d)


def kernel(hidden_states, classifier_weight, e_score_correction_bias):
    hs = hidden_states.reshape(-1, HIDDEN).astype(jnp.float32)
    bias2d = e_score_correction_bias.reshape(1, NUM_EXPERTS)
    idx, wgt = _router(hs, classifier_weight, bias2d)
    return idx, wgt
